# baseline (speedup 1.0000x reference)
import jax
import jax.numpy as jnp
from jax import lax
from jax.experimental import pallas as pl
from jax.experimental.pallas import tpu as pltpu
from jax.experimental.pallas import tpu_sc as plsc

N_USERS = 5000
N_NODES = 10000
EMB_DIM = 128
HALF_DIM = 64
N_EDGES = 320000
CHUNK = 128

DEG_E_TILE = (N_EDGES // 2) // 16
DEG_FULL = DEG_E_TILE // CHUNK
DEG_BATCH = 6
DEG_REM = DEG_E_TILE - DEG_FULL * CHUNK
DEG_ROWS = N_USERS // 16
DEG_TAIL = N_USERS - 16 * DEG_ROWS

SP_CH = 64
SP_BATCH = 12
SP_E_TILE = (N_EDGES // 2) // 16
SP_FULL = SP_E_TILE // SP_CH
SP_REM = SP_E_TILE - SP_FULL * SP_CH
SP_ROWS = N_USERS // 16
SP_TAIL = N_USERS - 16 * SP_ROWS

_MESH = plsc.VectorSubcoreMesh(
    core_axis_name="c", subcore_axis_name="s", num_cores=2, num_subcores=16
)


def _deg_body(h_hbm, deg_hbm, hbat, idxa, idxb, idx16_v, buf, ones16, acc,
              dsem_a, dsem_b):
    cid = lax.axis_index("c")
    sid = lax.axis_index("s")
    noff = cid * N_USERS

    @pl.loop(0, CHUNK)
    def _(i):
        for j in range(8):
            buf[i, pl.ds(16 * j, 16)] = jnp.zeros((16,), jnp.float32)

    zbase = sid * DEG_ROWS
    nlast = DEG_ROWS - 2 * CHUNK
    for r in range(2):
        pltpu.sync_copy(buf, acc.at[pl.ds(zbase + r * CHUNK, CHUNK)])
    pltpu.sync_copy(buf.at[pl.ds(0, nlast)],
                    acc.at[pl.ds(zbase + 2 * CHUNK, nlast)])

    @pl.when(sid == 0)
    def _():
        pltpu.sync_copy(buf.at[pl.ds(0, DEG_TAIL)],
                        acc.at[pl.ds(16 * DEG_ROWS, DEG_TAIL)])

    @pl.loop(0, CHUNK)
    def _(i):
        for j in range(8):
            buf[i, pl.ds(16 * j, 16)] = jnp.ones((16,), jnp.float32)

    @pl.loop(0, DEG_REM)
    def _(i):
        for j in range(8):
            ones16[i, pl.ds(16 * j, 16)] = jnp.ones((16,), jnp.float32)

    plsc.subcore_barrier()

    ebase = cid * (N_EDGES // 2) + sid * DEG_E_TILE

    def build_deg_idx(i_ref, hb_off):
        for i in range(8):
            i_ref[pl.ds(16 * i, 16)] = hbat[pl.ds(hb_off + 16 * i, 16)] - noff

    @pl.loop(0, DEG_FULL // DEG_BATCH)
    def _(bb):
        bbase = ebase + bb * DEG_BATCH * CHUNK
        pltpu.sync_copy(h_hbm.at[pl.ds(bbase, DEG_BATCH * CHUNK)], hbat)

        @pl.loop(0, DEG_BATCH, step=2)
        def _(j):
            build_deg_idx(idxa, j * CHUNK)
            pltpu.make_async_copy(buf, acc.at[idxa], dsem_a).start()
            build_deg_idx(idxb, (j + 1) * CHUNK)
            pltpu.make_async_copy(buf, acc.at[idxb], dsem_b).start()
            pltpu.make_async_copy(buf, acc.at[idxa], dsem_a).wait()
            pltpu.make_async_copy(buf, acc.at[idxb], dsem_b).wait()

    pltpu.sync_copy(h_hbm.at[pl.ds(ebase + DEG_FULL * CHUNK, DEG_REM)],
                    idx16_v)
    idx16_v[pl.ds(0, 16)] = idx16_v[pl.ds(0, 16)] - noff
    pltpu.sync_copy(ones16, acc.at[idx16_v], add=True)

    plsc.subcore_barrier()

    for r in range(2):
        pltpu.sync_copy(acc.at[pl.ds(zbase + r * CHUNK, CHUNK)], buf)
        pltpu.sync_copy(buf, deg_hbm.at[pl.ds(noff + zbase + r * CHUNK, CHUNK)])
    pltpu.sync_copy(acc.at[pl.ds(zbase + 2 * CHUNK, nlast)],
                    buf.at[pl.ds(0, nlast)])
    pltpu.sync_copy(buf.at[pl.ds(0, nlast)],
                    deg_hbm.at[pl.ds(noff + zbase + 2 * CHUNK, nlast)])

    @pl.when(sid == 0)
    def _():
        b = 16 * DEG_ROWS
        pltpu.sync_copy(acc.at[pl.ds(b, DEG_TAIL)], buf.at[pl.ds(0, DEG_TAIL)])
        pltpu.sync_copy(buf.at[pl.ds(0, DEG_TAIL)],
                        deg_hbm.at[pl.ds(noff + b, DEG_TAIL)])


def _sc_degree(h):
    return pl.kernel(
        _deg_body,
        out_type=jax.ShapeDtypeStruct((N_NODES, EMB_DIM), jnp.float32),
        mesh=_MESH,
        scratch_types=[
            pltpu.VMEM((DEG_BATCH * CHUNK,), jnp.int32),
            pltpu.VMEM((CHUNK,), jnp.int32),
            pltpu.VMEM((CHUNK,), jnp.int32),
            pltpu.VMEM((DEG_REM,), jnp.int32),
            pltpu.VMEM((CHUNK, EMB_DIM), jnp.float32),
            pltpu.VMEM((DEG_REM, EMB_DIM), jnp.float32),
            pltpu.VMEM_SHARED((N_USERS, EMB_DIM), jnp.float32),
            pltpu.SemaphoreType.DMA,
            pltpu.SemaphoreType.DMA,
        ],
    )(h)


def _spmm_body(z_hbm, t_hbm, h_hbm, y_hbm,
               tbatch, hbatch, ha, hb, t16, h16, rows_a, rows_b, rows16,
               acc, sem_a, sem_b, sem_sa, sem_sb):
    cid = lax.axis_index("c")
    sid = lax.axis_index("s")
    noff = cid * N_USERS

    @pl.loop(0, SP_CH)
    def _(i):
        for j in range(8):
            rows_a[i, pl.ds(16 * j, 16)] = jnp.zeros((16,), jnp.float32)

    zbase = sid * SP_ROWS
    nlast = SP_ROWS - 4 * SP_CH
    for r in range(4):
        pltpu.sync_copy(rows_a, acc.at[pl.ds(zbase + r * SP_CH, SP_CH)])
    pltpu.sync_copy(rows_a.at[pl.ds(0, nlast)],
                    acc.at[pl.ds(zbase + 4 * SP_CH, nlast)])

    @pl.when(sid == 0)
    def _():
        pltpu.sync_copy(rows_a.at[pl.ds(0, SP_TAIL)],
                        acc.at[pl.ds(16 * SP_ROWS, SP_TAIL)])

    plsc.subcore_barrier()

    ebase = cid * (N_EDGES // 2) + sid * SP_E_TILE

    def build_h(h_ref, j):
        for i in range(SP_CH // 16):
            h_ref[pl.ds(16 * i, 16)] = (
                hbatch[pl.ds(j * SP_CH + 16 * i, 16)] - noff)

    def gather(j, rows_ref, sem):
        idx = tbatch.at[pl.ds(j * SP_CH, SP_CH)]
        return pltpu.make_async_copy(z_hbm.at[idx], rows_ref, sem)

    @pl.loop(0, SP_FULL // SP_BATCH)
    def _(bb):
        bbase = ebase + bb * SP_BATCH * SP_CH
        pltpu.sync_copy(t_hbm.at[pl.ds(bbase, SP_BATCH * SP_CH)], tbatch)
        pltpu.sync_copy(h_hbm.at[pl.ds(bbase, SP_BATCH * SP_CH)], hbatch)
        gather(0, rows_a, sem_a).start()

        @pl.loop(0, SP_BATCH, step=2)
        def _(j):
            gather(j + 1, rows_b, sem_b).start()
            gather(j, rows_a, sem_a).wait()
            build_h(ha, j)
            scat_a = pltpu.make_async_copy(rows_a, acc.at[ha], sem_sa)
            scat_a.start()
            gather(j + 1, rows_b, sem_b).wait()
            build_h(hb, j + 1)
            scat_b = pltpu.make_async_copy(rows_b, acc.at[hb], sem_sb)
            scat_b.start()
            pltpu.make_async_copy(rows_a, acc.at[ha], sem_sa).wait()

            @pl.when(j + 2 < SP_BATCH)
            def _():
                gather(j + 2, rows_a, sem_a).start()

            pltpu.make_async_copy(rows_b, acc.at[hb], sem_sb).wait()

    b = ebase + SP_FULL * SP_CH
    pltpu.sync_copy(t_hbm.at[pl.ds(b, SP_REM)], t16)
    pltpu.sync_copy(h_hbm.at[pl.ds(b, SP_REM)], h16)
    h16[pl.ds(0, 16)] = h16[pl.ds(0, 16)] - noff
    pltpu.async_copy(z_hbm.at[t16], rows16, sem_a).wait()
    pltpu.sync_copy(rows16, acc.at[h16], add=True)

    plsc.subcore_barrier()

    for r in range(4):
        pltpu.sync_copy(acc.at[pl.ds(zbase + r * SP_CH, SP_CH)], rows_a)
        pltpu.sync_copy(rows_a, y_hbm.at[pl.ds(noff + zbase + r * SP_CH, SP_CH)])
    pltpu.sync_copy(acc.at[pl.ds(zbase + 4 * SP_CH, nlast)],
                    rows_a.at[pl.ds(0, nlast)])
    pltpu.sync_copy(rows_a.at[pl.ds(0, nlast)],
                    y_hbm.at[pl.ds(noff + zbase + 4 * SP_CH, nlast)])

    @pl.when(sid == 0)
    def _():
        b2 = 16 * SP_ROWS
        pltpu.sync_copy(acc.at[pl.ds(b2, SP_TAIL)], rows_a.at[pl.ds(0, SP_TAIL)])
        pltpu.sync_copy(rows_a.at[pl.ds(0, SP_TAIL)],
                        y_hbm.at[pl.ds(noff + b2, SP_TAIL)])


def _sc_spmm(z, t, h):
    return pl.kernel(
        _spmm_body,
        out_type=jax.ShapeDtypeStruct((N_NODES, EMB_DIM), jnp.float32),
        mesh=_MESH,
        scratch_types=[
            pltpu.VMEM((SP_BATCH * SP_CH,), jnp.int32),
            pltpu.VMEM((SP_BATCH * SP_CH,), jnp.int32),
            pltpu.VMEM((SP_CH,), jnp.int32),
            pltpu.VMEM((SP_CH,), jnp.int32),
            pltpu.VMEM((SP_REM,), jnp.int32),
            pltpu.VMEM((SP_REM,), jnp.int32),
            pltpu.VMEM((SP_CH, EMB_DIM), jnp.float32),
            pltpu.VMEM((SP_CH, EMB_DIM), jnp.float32),
            pltpu.VMEM((SP_REM, EMB_DIM), jnp.float32),
            pltpu.VMEM_SHARED((N_USERS, EMB_DIM), jnp.float32),
            pltpu.SemaphoreType.DMA,
            pltpu.SemaphoreType.DMA,
            pltpu.SemaphoreType.DMA,
            pltpu.SemaphoreType.DMA,
        ],
    )(z, t, h)


_BLK = 1000


def _dinv_of(deg_ref):
    d = deg_ref[:, 0:1]
    return jnp.where(d > 0, lax.rsqrt(d), 0.0)


def _intent_kernel(x_ref, w_ref, wt_ref, o_ref):
    p = jnp.dot(x_ref[...], w_ref[0], preferred_element_type=jnp.float32)
    p = p - jnp.max(p, axis=1, keepdims=True)
    e = jnp.exp(p)
    s = e / jnp.sum(e, axis=1, keepdims=True)
    o_ref[...] = jnp.dot(s, wt_ref[0], preferred_element_type=jnp.float32)


def _scale_kernel(deg_ref, x_ref, z_ref):
    z_ref[...] = x_ref[...] * _dinv_of(deg_ref)


def _combine1_kernel(deg_ref, y_ref, int_ref, x0_ref, gnn_ref, x1_ref, z1_ref):
    dinv = _dinv_of(deg_ref)
    g = dinv * y_ref[...]
    x1 = g + int_ref[...] + x0_ref[...]
    gnn_ref[...] = g
    x1_ref[...] = x1
    z1_ref[...] = dinv * x1


def _combine2_kernel(deg_ref, y_ref, int_ref, x0_ref, x1_ref,
                     gnn_ref, sum_ref):
    dinv = _dinv_of(deg_ref)
    g = dinv * y_ref[...]
    gnn_ref[...] = g
    sum_ref[...] = x0_ref[...] + 2.0 * x1_ref[...] + g + int_ref[...]


_ROW_SPEC = pl.BlockSpec((_BLK, EMB_DIM), lambda i: (i, 0))
_DEG_SPEC = pl.BlockSpec((_BLK, EMB_DIM), lambda i: (i, 0))
_W_SPEC = pl.BlockSpec((1, EMB_DIM, EMB_DIM), lambda i: (i // 5, 0, 0))
_EMB = jax.ShapeDtypeStruct((N_NODES, EMB_DIM), jnp.float32)


def _tc_intent(x, wb, wtb):
    return pl.pallas_call(
        _intent_kernel,
        grid=(N_NODES // _BLK,),
        in_specs=[_ROW_SPEC, _W_SPEC, _W_SPEC],
        out_specs=_ROW_SPEC,
        out_shape=_EMB,
    )(x, wb, wtb)


def _tc_scale(deg, x):
    return pl.pallas_call(
        _scale_kernel,
        grid=(N_NODES // _BLK,),
        in_specs=[_DEG_SPEC, _ROW_SPEC],
        out_specs=_ROW_SPEC,
        out_shape=_EMB,
    )(deg, x)


def _tc_combine1(deg, y, intv, x0):
    return pl.pallas_call(
        _combine1_kernel,
        grid=(N_NODES // _BLK,),
        in_specs=[_DEG_SPEC, _ROW_SPEC, _ROW_SPEC, _ROW_SPEC],
        out_specs=[_ROW_SPEC, _ROW_SPEC, _ROW_SPEC],
        out_shape=[_EMB, _EMB, _EMB],
    )(deg, y, intv, x0)


def _tc_combine2(deg, y, intv, x0, x1):
    return pl.pallas_call(
        _combine2_kernel,
        grid=(N_NODES // _BLK,),
        in_specs=[_DEG_SPEC, _ROW_SPEC, _ROW_SPEC, _ROW_SPEC, _ROW_SPEC],
        out_specs=[_ROW_SPEC, _ROW_SPEC],
        out_shape=[_EMB, _EMB],
    )(deg, y, intv, x0, x1)


def kernel(user_feat, item_feat, user_intent, item_intent, edge_index):
    h = edge_index[0].astype(jnp.int32)
    t = edge_index[1].astype(jnp.int32)
    x0 = jnp.concatenate([user_feat, item_feat], axis=0)
    wb = jnp.stack([user_intent, item_intent], axis=0)
    wtb = jnp.stack([user_intent.T, item_intent.T], axis=0)

    deg = _sc_degree(h)
    int1 = _tc_intent(x0, wb, wtb)
    z0 = _tc_scale(deg, x0)
    y0 = _sc_spmm(z0, t, h)
    gnn1, x1, z1 = _tc_combine1(deg, y0, int1, x0)
    int2 = _tc_intent(x1, wb, wtb)
    y1 = _sc_spmm(z1, t, h)
    gnn2, summ = _tc_combine2(deg, y1, int2, x0, x1)

    return (summ[:N_USERS], summ[N_USERS:], gnn1, gnn2, int1, int2)


# speedup vs baseline: 1.0890x; 1.0890x over previous
import jax
import jax.numpy as jnp
from jax import lax
from jax.experimental import pallas as pl
from jax.experimental.pallas import tpu as pltpu
from jax.experimental.pallas import tpu_sc as plsc

N_USERS = 5000
N_NODES = 10000
EMB_DIM = 128
HALF_DIM = 64
N_EDGES = 320000
CHUNK = 128

DEG_E_TILE = (N_EDGES // 2) // 16
DEG_FULL = DEG_E_TILE // CHUNK
DEG_BATCH = 6
DEG_REM = DEG_E_TILE - DEG_FULL * CHUNK
DEG_ROWS = N_USERS // 16
DEG_TAIL = N_USERS - 16 * DEG_ROWS

SP_CH = 64
SP_BATCH = 12
SP_E_TILE = (N_EDGES // 2) // 16
SP_FULL = SP_E_TILE // SP_CH
SP_REM = SP_E_TILE - SP_FULL * SP_CH
SP_ROWS = N_USERS // 16
SP_TAIL = N_USERS - 16 * SP_ROWS

_MESH = plsc.VectorSubcoreMesh(
    core_axis_name="c", subcore_axis_name="s", num_cores=2, num_subcores=16
)


def _deg_body(h_hbm, deg_hbm, hbat, idx_v, idx16_v, buf, ones16, acc):
    cid = lax.axis_index("c")
    sid = lax.axis_index("s")
    noff = cid * N_USERS

    @pl.loop(0, CHUNK)
    def _(i):
        for j in range(8):
            buf[i, pl.ds(16 * j, 16)] = jnp.zeros((16,), jnp.float32)

    zbase = sid * DEG_ROWS
    nlast = DEG_ROWS - 2 * CHUNK
    for r in range(2):
        pltpu.sync_copy(buf, acc.at[pl.ds(zbase + r * CHUNK, CHUNK)])
    pltpu.sync_copy(buf.at[pl.ds(0, nlast)],
                    acc.at[pl.ds(zbase + 2 * CHUNK, nlast)])

    @pl.when(sid == 0)
    def _():
        pltpu.sync_copy(buf.at[pl.ds(0, DEG_TAIL)],
                        acc.at[pl.ds(16 * DEG_ROWS, DEG_TAIL)])

    @pl.loop(0, CHUNK)
    def _(i):
        for j in range(8):
            buf[i, pl.ds(16 * j, 16)] = jnp.ones((16,), jnp.float32)

    @pl.loop(0, DEG_REM)
    def _(i):
        for j in range(8):
            ones16[i, pl.ds(16 * j, 16)] = jnp.ones((16,), jnp.float32)

    plsc.subcore_barrier()

    ebase = cid * (N_EDGES // 2) + sid * DEG_E_TILE

    @pl.loop(0, DEG_FULL // DEG_BATCH)
    def _(bb):
        bbase = ebase + bb * DEG_BATCH * CHUNK
        pltpu.sync_copy(h_hbm.at[pl.ds(bbase, DEG_BATCH * CHUNK)], hbat)

        @pl.loop(0, DEG_BATCH)
        def _(k):
            for j in range(8):
                idx_v[pl.ds(16 * j, 16)] = (
                    hbat[pl.ds(k * CHUNK + 16 * j, 16)] - noff)
            pltpu.sync_copy(buf, acc.at[idx_v], add=True)

    pltpu.sync_copy(h_hbm.at[pl.ds(ebase + DEG_FULL * CHUNK, DEG_REM)],
                    idx16_v)
    idx16_v[pl.ds(0, 16)] = idx16_v[pl.ds(0, 16)] - noff
    pltpu.sync_copy(ones16, acc.at[idx16_v], add=True)

    plsc.subcore_barrier()

    for r in range(2):
        pltpu.sync_copy(acc.at[pl.ds(zbase + r * CHUNK, CHUNK)], buf)
        pltpu.sync_copy(buf, deg_hbm.at[pl.ds(noff + zbase + r * CHUNK, CHUNK)])
    pltpu.sync_copy(acc.at[pl.ds(zbase + 2 * CHUNK, nlast)],
                    buf.at[pl.ds(0, nlast)])
    pltpu.sync_copy(buf.at[pl.ds(0, nlast)],
                    deg_hbm.at[pl.ds(noff + zbase + 2 * CHUNK, nlast)])

    @pl.when(sid == 0)
    def _():
        b = 16 * DEG_ROWS
        pltpu.sync_copy(acc.at[pl.ds(b, DEG_TAIL)], buf.at[pl.ds(0, DEG_TAIL)])
        pltpu.sync_copy(buf.at[pl.ds(0, DEG_TAIL)],
                        deg_hbm.at[pl.ds(noff + b, DEG_TAIL)])


def _sc_degree(h):
    return pl.kernel(
        _deg_body,
        out_type=jax.ShapeDtypeStruct((N_NODES, EMB_DIM), jnp.float32),
        mesh=_MESH,
        scratch_types=[
            pltpu.VMEM((DEG_BATCH * CHUNK,), jnp.int32),
            pltpu.VMEM((CHUNK,), jnp.int32),
            pltpu.VMEM((DEG_REM,), jnp.int32),
            pltpu.VMEM((CHUNK, EMB_DIM), jnp.float32),
            pltpu.VMEM((DEG_REM, EMB_DIM), jnp.float32),
            pltpu.VMEM_SHARED((N_USERS, EMB_DIM), jnp.float32),
        ],
    )(h)


def _spmm_body(z_hbm, t_hbm, h_hbm, y_hbm,
               tbatch, hbatch, ha, hb, t16, h16, rows_a, rows_b, rows16,
               acc, sem_a, sem_b):
    cid = lax.axis_index("c")
    sid = lax.axis_index("s")
    noff = cid * N_USERS

    @pl.loop(0, SP_CH)
    def _(i):
        for j in range(8):
            rows_a[i, pl.ds(16 * j, 16)] = jnp.zeros((16,), jnp.float32)

    zbase = sid * SP_ROWS
    nlast = SP_ROWS - 4 * SP_CH
    for r in range(4):
        pltpu.sync_copy(rows_a, acc.at[pl.ds(zbase + r * SP_CH, SP_CH)])
    pltpu.sync_copy(rows_a.at[pl.ds(0, nlast)],
                    acc.at[pl.ds(zbase + 4 * SP_CH, nlast)])

    @pl.when(sid == 0)
    def _():
        pltpu.sync_copy(rows_a.at[pl.ds(0, SP_TAIL)],
                        acc.at[pl.ds(16 * SP_ROWS, SP_TAIL)])

    plsc.subcore_barrier()

    ebase = cid * (N_EDGES // 2) + sid * SP_E_TILE

    def build_h(h_ref, j):
        for i in range(SP_CH // 16):
            h_ref[pl.ds(16 * i, 16)] = (
                hbatch[pl.ds(j * SP_CH + 16 * i, 16)] - noff)

    def gather(j, rows_ref, sem):
        idx = tbatch.at[pl.ds(j * SP_CH, SP_CH)]
        return pltpu.make_async_copy(z_hbm.at[idx], rows_ref, sem)

    @pl.loop(0, SP_FULL // SP_BATCH)
    def _(bb):
        bbase = ebase + bb * SP_BATCH * SP_CH
        pltpu.sync_copy(t_hbm.at[pl.ds(bbase, SP_BATCH * SP_CH)], tbatch)
        pltpu.sync_copy(h_hbm.at[pl.ds(bbase, SP_BATCH * SP_CH)], hbatch)
        gather(0, rows_a, sem_a).start()

        @pl.loop(0, SP_BATCH, step=2)
        def _(j):
            gather(j + 1, rows_b, sem_b).start()
            gather(j, rows_a, sem_a).wait()
            build_h(ha, j)
            pltpu.sync_copy(rows_a, acc.at[ha], add=True)

            @pl.when(j + 2 < SP_BATCH)
            def _():
                gather(j + 2, rows_a, sem_a).start()

            gather(j + 1, rows_b, sem_b).wait()
            build_h(hb, j + 1)
            pltpu.sync_copy(rows_b, acc.at[hb], add=True)

    b = ebase + SP_FULL * SP_CH
    pltpu.sync_copy(t_hbm.at[pl.ds(b, SP_REM)], t16)
    pltpu.sync_copy(h_hbm.at[pl.ds(b, SP_REM)], h16)
    h16[pl.ds(0, 16)] = h16[pl.ds(0, 16)] - noff
    pltpu.async_copy(z_hbm.at[t16], rows16, sem_a).wait()
    pltpu.sync_copy(rows16, acc.at[h16], add=True)

    plsc.subcore_barrier()

    for r in range(4):
        pltpu.sync_copy(acc.at[pl.ds(zbase + r * SP_CH, SP_CH)], rows_a)
        pltpu.sync_copy(rows_a, y_hbm.at[pl.ds(noff + zbase + r * SP_CH, SP_CH)])
    pltpu.sync_copy(acc.at[pl.ds(zbase + 4 * SP_CH, nlast)],
                    rows_a.at[pl.ds(0, nlast)])
    pltpu.sync_copy(rows_a.at[pl.ds(0, nlast)],
                    y_hbm.at[pl.ds(noff + zbase + 4 * SP_CH, nlast)])

    @pl.when(sid == 0)
    def _():
        b2 = 16 * SP_ROWS
        pltpu.sync_copy(acc.at[pl.ds(b2, SP_TAIL)], rows_a.at[pl.ds(0, SP_TAIL)])
        pltpu.sync_copy(rows_a.at[pl.ds(0, SP_TAIL)],
                        y_hbm.at[pl.ds(noff + b2, SP_TAIL)])


def _sc_spmm(z, t, h):
    return pl.kernel(
        _spmm_body,
        out_type=jax.ShapeDtypeStruct((N_NODES, EMB_DIM), jnp.float32),
        mesh=_MESH,
        scratch_types=[
            pltpu.VMEM((SP_BATCH * SP_CH,), jnp.int32),
            pltpu.VMEM((SP_BATCH * SP_CH,), jnp.int32),
            pltpu.VMEM((SP_CH,), jnp.int32),
            pltpu.VMEM((SP_CH,), jnp.int32),
            pltpu.VMEM((SP_REM,), jnp.int32),
            pltpu.VMEM((SP_REM,), jnp.int32),
            pltpu.VMEM((SP_CH, EMB_DIM), jnp.float32),
            pltpu.VMEM((SP_CH, EMB_DIM), jnp.float32),
            pltpu.VMEM((SP_REM, EMB_DIM), jnp.float32),
            pltpu.VMEM_SHARED((N_USERS, EMB_DIM), jnp.float32),
            pltpu.SemaphoreType.DMA,
            pltpu.SemaphoreType.DMA,
        ],
    )(z, t, h)


_BLK = 1000


def _dinv_of(deg_ref):
    d = deg_ref[:, 0:1]
    return jnp.where(d > 0, lax.rsqrt(d), 0.0)


def _intent_kernel(x_ref, w_ref, wt_ref, o_ref):
    p = jnp.dot(x_ref[...], w_ref[0], preferred_element_type=jnp.float32)
    p = p - jnp.max(p, axis=1, keepdims=True)
    e = jnp.exp(p)
    s = e / jnp.sum(e, axis=1, keepdims=True)
    o_ref[...] = jnp.dot(s, wt_ref[0], preferred_element_type=jnp.float32)


def _scale_kernel(deg_ref, x_ref, z_ref):
    z_ref[...] = x_ref[...] * _dinv_of(deg_ref)


def _combine1_kernel(deg_ref, y_ref, int_ref, x0_ref, gnn_ref, x1_ref, z1_ref):
    dinv = _dinv_of(deg_ref)
    g = dinv * y_ref[...]
    x1 = g + int_ref[...] + x0_ref[...]
    gnn_ref[...] = g
    x1_ref[...] = x1
    z1_ref[...] = dinv * x1


def _combine2_kernel(deg_ref, y_ref, int_ref, x0_ref, x1_ref,
                     gnn_ref, sum_ref):
    dinv = _dinv_of(deg_ref)
    g = dinv * y_ref[...]
    gnn_ref[...] = g
    sum_ref[...] = x0_ref[...] + 2.0 * x1_ref[...] + g + int_ref[...]


_ROW_SPEC = pl.BlockSpec((_BLK, EMB_DIM), lambda i: (i, 0))
_DEG_SPEC = pl.BlockSpec((_BLK, EMB_DIM), lambda i: (i, 0))
_W_SPEC = pl.BlockSpec((1, EMB_DIM, EMB_DIM), lambda i: (i // 5, 0, 0))
_EMB = jax.ShapeDtypeStruct((N_NODES, EMB_DIM), jnp.float32)


def _tc_intent(x, wb, wtb):
    return pl.pallas_call(
        _intent_kernel,
        grid=(N_NODES // _BLK,),
        in_specs=[_ROW_SPEC, _W_SPEC, _W_SPEC],
        out_specs=_ROW_SPEC,
        out_shape=_EMB,
    )(x, wb, wtb)


def _tc_scale(deg, x):
    return pl.pallas_call(
        _scale_kernel,
        grid=(N_NODES // _BLK,),
        in_specs=[_DEG_SPEC, _ROW_SPEC],
        out_specs=_ROW_SPEC,
        out_shape=_EMB,
    )(deg, x)


def _tc_combine1(deg, y, intv, x0):
    return pl.pallas_call(
        _combine1_kernel,
        grid=(N_NODES // _BLK,),
        in_specs=[_DEG_SPEC, _ROW_SPEC, _ROW_SPEC, _ROW_SPEC],
        out_specs=[_ROW_SPEC, _ROW_SPEC, _ROW_SPEC],
        out_shape=[_EMB, _EMB, _EMB],
    )(deg, y, intv, x0)


def _tc_combine2(deg, y, intv, x0, x1):
    return pl.pallas_call(
        _combine2_kernel,
        grid=(N_NODES // _BLK,),
        in_specs=[_DEG_SPEC, _ROW_SPEC, _ROW_SPEC, _ROW_SPEC, _ROW_SPEC],
        out_specs=[_ROW_SPEC, _ROW_SPEC],
        out_shape=[_EMB, _EMB],
    )(deg, y, intv, x0, x1)


def kernel(user_feat, item_feat, user_intent, item_intent, edge_index):
    h = edge_index[0].astype(jnp.int32)
    t = edge_index[1].astype(jnp.int32)
    x0 = jnp.concatenate([user_feat, item_feat], axis=0)
    wb = jnp.stack([user_intent, item_intent], axis=0)
    wtb = jnp.stack([user_intent.T, item_intent.T], axis=0)

    deg = _sc_degree(h)
    int1 = _tc_intent(x0, wb, wtb)
    z0 = _tc_scale(deg, x0)
    y0 = _sc_spmm(z0, t, h)
    gnn1, x1, z1 = _tc_combine1(deg, y0, int1, x0)
    int2 = _tc_intent(x1, wb, wtb)
    y1 = _sc_spmm(z1, t, h)
    gnn2, summ = _tc_combine2(deg, y1, int2, x0, x1)

    return (summ[:N_USERS], summ[N_USERS:], gnn1, gnn2, int1, int2)


# speedup vs baseline: 1.0891x; 1.0002x over previous
import jax
import jax.numpy as jnp
from jax import lax
from jax.experimental import pallas as pl
from jax.experimental.pallas import tpu as pltpu
from jax.experimental.pallas import tpu_sc as plsc

N_USERS = 5000
N_NODES = 10000
EMB_DIM = 128
N_EDGES = 320000
CHUNK = 128

DEG_E_TILE = (N_EDGES // 2) // 16
DEG_FULL = DEG_E_TILE // CHUNK
DEG_BATCH = 6
DEG_REM = DEG_E_TILE - DEG_FULL * CHUNK
DEG_ROWS = N_USERS // 16
DEG_TAIL = N_USERS - 16 * DEG_ROWS

SP_CH = 64
SP_BATCH = 12
SP_E_TILE = (N_EDGES // 2) // 16
SP_FULL = SP_E_TILE // SP_CH
SP_REM = SP_E_TILE - SP_FULL * SP_CH
SP_ROWS = N_USERS // 16
SP_TAIL = N_USERS - 16 * SP_ROWS

_MESH = plsc.VectorSubcoreMesh(
    core_axis_name="c", subcore_axis_name="s", num_cores=2, num_subcores=16
)


def _deg_body(h_hbm, deg_hbm, hbat, idx_v, idx16_v, buf, ones16, acc):
    cid = lax.axis_index("c")
    sid = lax.axis_index("s")
    noff = cid * N_USERS

    @pl.loop(0, CHUNK)
    def _(i):
        for j in range(8):
            buf[i, pl.ds(16 * j, 16)] = jnp.zeros((16,), jnp.float32)

    zbase = sid * DEG_ROWS
    nlast = DEG_ROWS - 2 * CHUNK
    for r in range(2):
        pltpu.sync_copy(buf, acc.at[pl.ds(zbase + r * CHUNK, CHUNK)])
    pltpu.sync_copy(buf.at[pl.ds(0, nlast)],
                    acc.at[pl.ds(zbase + 2 * CHUNK, nlast)])

    @pl.when(sid == 0)
    def _():
        pltpu.sync_copy(buf.at[pl.ds(0, DEG_TAIL)],
                        acc.at[pl.ds(16 * DEG_ROWS, DEG_TAIL)])

    @pl.loop(0, CHUNK)
    def _(i):
        for j in range(8):
            buf[i, pl.ds(16 * j, 16)] = jnp.ones((16,), jnp.float32)

    @pl.loop(0, DEG_REM)
    def _(i):
        for j in range(8):
            ones16[i, pl.ds(16 * j, 16)] = jnp.ones((16,), jnp.float32)

    plsc.subcore_barrier()

    ebase = cid * (N_EDGES // 2) + sid * DEG_E_TILE

    @pl.loop(0, DEG_FULL // DEG_BATCH)
    def _(bb):
        bbase = ebase + bb * DEG_BATCH * CHUNK
        pltpu.sync_copy(h_hbm.at[pl.ds(bbase, DEG_BATCH * CHUNK)], hbat)

        @pl.loop(0, DEG_BATCH)
        def _(k):
            for j in range(8):
                idx_v[pl.ds(16 * j, 16)] = (
                    hbat[pl.ds(k * CHUNK + 16 * j, 16)] - noff)
            pltpu.sync_copy(buf, acc.at[idx_v], add=True)

    pltpu.sync_copy(h_hbm.at[pl.ds(ebase + DEG_FULL * CHUNK, DEG_REM)],
                    idx16_v)
    idx16_v[pl.ds(0, 16)] = idx16_v[pl.ds(0, 16)] - noff
    pltpu.sync_copy(ones16, acc.at[idx16_v], add=True)

    plsc.subcore_barrier()

    for r in range(2):
        pltpu.sync_copy(acc.at[pl.ds(zbase + r * CHUNK, CHUNK)], buf)
        pltpu.sync_copy(buf, deg_hbm.at[pl.ds(noff + zbase + r * CHUNK, CHUNK)])
    pltpu.sync_copy(acc.at[pl.ds(zbase + 2 * CHUNK, nlast)],
                    buf.at[pl.ds(0, nlast)])
    pltpu.sync_copy(buf.at[pl.ds(0, nlast)],
                    deg_hbm.at[pl.ds(noff + zbase + 2 * CHUNK, nlast)])

    @pl.when(sid == 0)
    def _():
        b = 16 * DEG_ROWS
        pltpu.sync_copy(acc.at[pl.ds(b, DEG_TAIL)], buf.at[pl.ds(0, DEG_TAIL)])
        pltpu.sync_copy(buf.at[pl.ds(0, DEG_TAIL)],
                        deg_hbm.at[pl.ds(noff + b, DEG_TAIL)])


def _sc_degree(h):
    return pl.kernel(
        _deg_body,
        out_type=jax.ShapeDtypeStruct((N_NODES, EMB_DIM), jnp.float32),
        mesh=_MESH,
        scratch_types=[
            pltpu.VMEM((DEG_BATCH * CHUNK,), jnp.int32),
            pltpu.VMEM((CHUNK,), jnp.int32),
            pltpu.VMEM((DEG_REM,), jnp.int32),
            pltpu.VMEM((CHUNK, EMB_DIM), jnp.float32),
            pltpu.VMEM((DEG_REM, EMB_DIM), jnp.float32),
            pltpu.VMEM_SHARED((N_USERS, EMB_DIM), jnp.float32),
        ],
    )(h)


def _spmm_body(z_hbm, t_hbm, h_hbm, y_hbm,
               tbatch, hbatch, ha, hb, t16, h16, rows_a, rows_b, rows16,
               acc, sem_a, sem_b):
    cid = lax.axis_index("c")
    sid = lax.axis_index("s")
    noff = cid * N_USERS

    @pl.loop(0, SP_CH)
    def _(i):
        for j in range(8):
            rows_a[i, pl.ds(16 * j, 16)] = jnp.zeros((16,), jnp.float32)

    zbase = sid * SP_ROWS
    nlast = SP_ROWS - 4 * SP_CH
    for r in range(4):
        pltpu.sync_copy(rows_a, acc.at[pl.ds(zbase + r * SP_CH, SP_CH)])
    pltpu.sync_copy(rows_a.at[pl.ds(0, nlast)],
                    acc.at[pl.ds(zbase + 4 * SP_CH, nlast)])

    @pl.when(sid == 0)
    def _():
        pltpu.sync_copy(rows_a.at[pl.ds(0, SP_TAIL)],
                        acc.at[pl.ds(16 * SP_ROWS, SP_TAIL)])

    plsc.subcore_barrier()

    ebase = cid * (N_EDGES // 2) + sid * SP_E_TILE

    def build_h(h_ref, j):
        for i in range(SP_CH // 16):
            h_ref[pl.ds(16 * i, 16)] = (
                hbatch[pl.ds(j * SP_CH + 16 * i, 16)] - noff)

    def gather(j, rows_ref, sem):
        idx = tbatch.at[pl.ds(j * SP_CH, SP_CH)]
        return pltpu.make_async_copy(z_hbm.at[idx], rows_ref, sem)

    @pl.loop(0, SP_FULL // SP_BATCH)
    def _(bb):
        bbase = ebase + bb * SP_BATCH * SP_CH
        pltpu.sync_copy(t_hbm.at[pl.ds(bbase, SP_BATCH * SP_CH)], tbatch)
        pltpu.sync_copy(h_hbm.at[pl.ds(bbase, SP_BATCH * SP_CH)], hbatch)
        gather(0, rows_a, sem_a).start()

        @pl.loop(0, SP_BATCH, step=2)
        def _(j):
            gather(j + 1, rows_b, sem_b).start()
            gather(j, rows_a, sem_a).wait()
            build_h(ha, j)
            pltpu.sync_copy(rows_a, acc.at[ha], add=True)

            @pl.when(j + 2 < SP_BATCH)
            def _():
                gather(j + 2, rows_a, sem_a).start()

            gather(j + 1, rows_b, sem_b).wait()
            build_h(hb, j + 1)
            pltpu.sync_copy(rows_b, acc.at[hb], add=True)

    b = ebase + SP_FULL * SP_CH
    pltpu.sync_copy(t_hbm.at[pl.ds(b, SP_REM)], t16)
    pltpu.sync_copy(h_hbm.at[pl.ds(b, SP_REM)], h16)
    h16[pl.ds(0, 16)] = h16[pl.ds(0, 16)] - noff
    pltpu.async_copy(z_hbm.at[t16], rows16, sem_a).wait()
    pltpu.sync_copy(rows16, acc.at[h16], add=True)

    plsc.subcore_barrier()

    for r in range(4):
        pltpu.sync_copy(acc.at[pl.ds(zbase + r * SP_CH, SP_CH)], rows_a)
        pltpu.sync_copy(rows_a, y_hbm.at[pl.ds(noff + zbase + r * SP_CH, SP_CH)])
    pltpu.sync_copy(acc.at[pl.ds(zbase + 4 * SP_CH, nlast)],
                    rows_a.at[pl.ds(0, nlast)])
    pltpu.sync_copy(rows_a.at[pl.ds(0, nlast)],
                    y_hbm.at[pl.ds(noff + zbase + 4 * SP_CH, nlast)])

    @pl.when(sid == 0)
    def _():
        b2 = 16 * SP_ROWS
        pltpu.sync_copy(acc.at[pl.ds(b2, SP_TAIL)], rows_a.at[pl.ds(0, SP_TAIL)])
        pltpu.sync_copy(rows_a.at[pl.ds(0, SP_TAIL)],
                        y_hbm.at[pl.ds(noff + b2, SP_TAIL)])


def _sc_spmm(z, t, h):
    return pl.kernel(
        _spmm_body,
        out_type=jax.ShapeDtypeStruct((N_NODES, EMB_DIM), jnp.float32),
        mesh=_MESH,
        scratch_types=[
            pltpu.VMEM((SP_BATCH * SP_CH,), jnp.int32),
            pltpu.VMEM((SP_BATCH * SP_CH,), jnp.int32),
            pltpu.VMEM((SP_CH,), jnp.int32),
            pltpu.VMEM((SP_CH,), jnp.int32),
            pltpu.VMEM((SP_REM,), jnp.int32),
            pltpu.VMEM((SP_REM,), jnp.int32),
            pltpu.VMEM((SP_CH, EMB_DIM), jnp.float32),
            pltpu.VMEM((SP_CH, EMB_DIM), jnp.float32),
            pltpu.VMEM((SP_REM, EMB_DIM), jnp.float32),
            pltpu.VMEM_SHARED((N_USERS, EMB_DIM), jnp.float32),
            pltpu.SemaphoreType.DMA,
            pltpu.SemaphoreType.DMA,
        ],
    )(z, t, h)


_BLK = 1000


def _dinv_of(deg_ref):
    d = deg_ref[:, 0:1]
    return jnp.where(d > 0, lax.rsqrt(d), 0.0)


def _intent_kernel(x_ref, w_ref, wt_ref, o_ref):
    p = jnp.dot(x_ref[...], w_ref[0], preferred_element_type=jnp.float32)
    p = p - jnp.max(p, axis=1, keepdims=True)
    e = jnp.exp(p)
    s = e / jnp.sum(e, axis=1, keepdims=True)
    o_ref[...] = jnp.dot(s, wt_ref[0], preferred_element_type=jnp.float32)


def _scale_kernel(deg_ref, x_ref, z_ref):
    z_ref[...] = x_ref[...] * _dinv_of(deg_ref)


def _combine1_kernel(deg_ref, y_ref, int_ref, x0_ref, gnn_ref, x1_ref, z1_ref):
    dinv = _dinv_of(deg_ref)
    g = dinv * y_ref[...]
    x1 = g + int_ref[...] + x0_ref[...]
    gnn_ref[...] = g
    x1_ref[...] = x1
    z1_ref[...] = dinv * x1


def _combine2_kernel(deg_ref, y_ref, int_ref, x0_ref, x1_ref,
                     gnn_ref, sum_ref):
    dinv = _dinv_of(deg_ref)
    g = dinv * y_ref[...]
    gnn_ref[...] = g
    sum_ref[...] = x0_ref[...] + 2.0 * x1_ref[...] + g + int_ref[...]


_ROW_SPEC = pl.BlockSpec((_BLK, EMB_DIM), lambda i: (i, 0))
_DEG_SPEC = pl.BlockSpec((_BLK, EMB_DIM), lambda i: (i, 0))
_W_SPEC = pl.BlockSpec((1, EMB_DIM, EMB_DIM), lambda i: (i // 5, 0, 0))
_EMB = jax.ShapeDtypeStruct((N_NODES, EMB_DIM), jnp.float32)


def _tc_intent(x, wb, wtb):
    return pl.pallas_call(
        _intent_kernel,
        grid=(N_NODES // _BLK,),
        in_specs=[_ROW_SPEC, _W_SPEC, _W_SPEC],
        out_specs=_ROW_SPEC,
        out_shape=_EMB,
    )(x, wb, wtb)


def _tc_scale(deg, x):
    return pl.pallas_call(
        _scale_kernel,
        grid=(N_NODES // _BLK,),
        in_specs=[_DEG_SPEC, _ROW_SPEC],
        out_specs=_ROW_SPEC,
        out_shape=_EMB,
    )(deg, x)


def _tc_combine1(deg, y, intv, x0):
    return pl.pallas_call(
        _combine1_kernel,
        grid=(N_NODES // _BLK,),
        in_specs=[_DEG_SPEC, _ROW_SPEC, _ROW_SPEC, _ROW_SPEC],
        out_specs=[_ROW_SPEC, _ROW_SPEC, _ROW_SPEC],
        out_shape=[_EMB, _EMB, _EMB],
    )(deg, y, intv, x0)


def _tc_combine2(deg, y, intv, x0, x1):
    return pl.pallas_call(
        _combine2_kernel,
        grid=(N_NODES // _BLK,),
        in_specs=[_DEG_SPEC, _ROW_SPEC, _ROW_SPEC, _ROW_SPEC, _ROW_SPEC],
        out_specs=[_ROW_SPEC, _ROW_SPEC],
        out_shape=[_EMB, _EMB],
    )(deg, y, intv, x0, x1)


def kernel(user_feat, item_feat, user_intent, item_intent, edge_index):
    h = edge_index[0].astype(jnp.int32)
    t = edge_index[1].astype(jnp.int32)
    x0 = jnp.concatenate([user_feat, item_feat], axis=0)
    wb = jnp.stack([user_intent, item_intent], axis=0)
    wtb = jnp.stack([user_intent.T, item_intent.T], axis=0)

    deg = _sc_degree(h)
    int1 = _tc_intent(x0, wb, wtb)
    z0 = _tc_scale(deg, x0)
    y0 = _sc_spmm(z0, t, h)
    gnn1, x1, z1 = _tc_combine1(deg, y0, int1, x0)
    int2 = _tc_intent(x1, wb, wtb)
    y1 = _sc_spmm(z1, t, h)
    gnn2, summ = _tc_combine2(deg, y1, int2, x0, x1)

    return (summ[:N_USERS], summ[N_USERS:], gnn1, gnn2, int1, int2)
